# baseline (speedup 1.0000x reference)
import functools

import jax
import jax.numpy as jnp
from jax import lax
from jax.experimental import pallas as pl
from jax.experimental.pallas import tpu as pltpu
from jax.experimental.pallas import tpu_sc as plsc

N = 10000
E = 320000
D = 128
G = 128

NP = 10240
SB = 128
NC, NS = 2, 16
NW = NC * NS
TPT = 80
EPAD = NW * TPT * SB
RPT = NP // NS
CROWS = 256
CPT = CROWS // NS
BPT = NP // NS // SB

RB = 1024
GRID = NP // RB

_mesh = plsc.VectorSubcoreMesh(core_axis_name="c", subcore_axis_name="s")


@functools.partial(
    pl.kernel,
    out_type=[
        jax.ShapeDtypeStruct((NC, NP, D), jnp.float32),
        jax.ShapeDtypeStruct((CROWS, D), jnp.float32),
    ],
    mesh=_mesh,
    scratch_types=[
        pltpu.VMEM_SHARED((NP, D), jnp.float32),
        pltpu.VMEM_SHARED((CROWS, D), jnp.float32),
        pltpu.VMEM((TPT, SB), jnp.int32),
        pltpu.VMEM((BPT, SB), jnp.int32),
        pltpu.VMEM((SB, D), jnp.float32),
        pltpu.SemaphoreType.DMA,
        pltpu.SemaphoreType.DMA,
    ],
)
def _sc_degrees(col3, batch3, ones128, zeros128, deg_out, cnt_out,
                dacc, cacc, colidx, batchidx, ones_v, ssem0, ssem1):
    c = lax.axis_index("c")
    s = lax.axis_index("s")
    wid = c * NS + s
    pltpu.sync_copy(zeros128.at[pl.ds(s * RPT, RPT)], dacc.at[pl.ds(s * RPT, RPT)])
    pltpu.sync_copy(zeros128.at[pl.ds(s * CPT, CPT)], cacc.at[pl.ds(s * CPT, CPT)])
    pltpu.sync_copy(ones128, ones_v)
    pltpu.sync_copy(col3.at[wid], colidx)
    plsc.subcore_barrier()

    def deg_pair(k2, carry):
        k = 2 * k2
        pltpu.async_copy(ones_v, dacc.at[colidx.at[k]], ssem0, add=True)

        @pl.when(k2 > 0)
        def _():
            pltpu.make_async_copy(ones_v, dacc.at[colidx.at[k - 1]],
                                  ssem1).wait()
        pltpu.async_copy(ones_v, dacc.at[colidx.at[k + 1]], ssem1, add=True)
        pltpu.make_async_copy(ones_v, dacc.at[colidx.at[k]], ssem0).wait()
        return carry
    lax.fori_loop(0, TPT // 2, deg_pair, 0)
    pltpu.make_async_copy(ones_v, dacc.at[colidx.at[TPT - 1]], ssem1).wait()

    @pl.when(c == 0)
    def _():
        pltpu.sync_copy(batch3.at[s], batchidx)

        def cnt_step(j, carry):
            pltpu.sync_copy(ones_v, cacc.at[batchidx.at[j]], add=True)
            return carry
        lax.fori_loop(0, BPT, cnt_step, 0)

    plsc.subcore_barrier()
    pltpu.sync_copy(dacc.at[pl.ds(s * RPT, RPT)],
                    deg_out.at[c, pl.ds(s * RPT, RPT)])

    @pl.when(c == 0)
    def _():
        pltpu.sync_copy(cacc.at[pl.ds(s * CPT, CPT)],
                        cnt_out.at[pl.ds(s * CPT, CPT)])


@functools.partial(
    pl.kernel,
    out_type=jax.ShapeDtypeStruct((NC, NP, D), jnp.float32),
    mesh=_mesh,
    scratch_types=[
        pltpu.VMEM_SHARED((NP, D), jnp.float32),
        pltpu.VMEM((TPT // 2, SB), jnp.int32),
        pltpu.VMEM((TPT // 2, SB), jnp.int32),
        pltpu.VMEM((SB, D), jnp.float32),
        pltpu.VMEM((SB, D), jnp.float32),
        pltpu.SemaphoreType.DMA,
        pltpu.SemaphoreType.DMA,
        pltpu.SemaphoreType.DMA,
        pltpu.SemaphoreType.DMA,
    ],
)
def _sc_aggregate(g_hbm, row3, col3, zeros128, out,
                  accum, rowidx, colidx, gb0, gb1, gsem0, gsem1, ssem0, ssem1):
    c = lax.axis_index("c")
    s = lax.axis_index("s")
    wid = c * NS + s
    HALF = TPT // 2
    pltpu.sync_copy(zeros128.at[pl.ds(s * RPT, RPT)],
                    accum.at[pl.ds(s * RPT, RPT)])
    plsc.subcore_barrier()

    for h in range(2):
        pltpu.sync_copy(row3.at[wid, pl.ds(h * HALF, HALF)], rowidx)
        pltpu.sync_copy(col3.at[wid, pl.ds(h * HALF, HALF)], colidx)
        pltpu.async_copy(g_hbm.at[rowidx.at[0]], gb0, gsem0)

        def pair(k2, carry):
            k = 2 * k2
            pltpu.make_async_copy(g_hbm.at[rowidx.at[k]], gb0, gsem0).wait()
            pltpu.async_copy(gb0, accum.at[colidx.at[k]], ssem0, add=True)

            @pl.when(k2 > 0)
            def _():
                pltpu.make_async_copy(gb1, accum.at[colidx.at[k - 1]],
                                      ssem1).wait()
            pltpu.async_copy(g_hbm.at[rowidx.at[k + 1]], gb1, gsem1)

            pltpu.make_async_copy(g_hbm.at[rowidx.at[k + 1]], gb1, gsem1).wait()
            pltpu.async_copy(gb1, accum.at[colidx.at[k + 1]], ssem1, add=True)
            pltpu.make_async_copy(gb0, accum.at[colidx.at[k]], ssem0).wait()

            @pl.when(k + 2 < HALF)
            def _():
                pltpu.async_copy(g_hbm.at[rowidx.at[k + 2]], gb0, gsem0)
            return carry
        lax.fori_loop(0, HALF // 2, pair, 0)
        pltpu.make_async_copy(gb1, accum.at[colidx.at[HALF - 1]], ssem1).wait()

    plsc.subcore_barrier()
    pltpu.sync_copy(accum.at[pl.ds(s * RPT, RPT)],
                    out.at[c, pl.ds(s * RPT, RPT)])


@functools.partial(
    pl.kernel,
    out_type=jax.ShapeDtypeStruct((CROWS, D), jnp.float32),
    mesh=_mesh,
    scratch_types=[
        pltpu.VMEM_SHARED((CROWS, D), jnp.float32),
        pltpu.VMEM((BPT, SB), jnp.int32),
        pltpu.VMEM((SB, D), jnp.float32),
    ],
)
def _sc_pool(f_hbm, batch3, zeros128, out, pacc, batchidx, rb):
    c = lax.axis_index("c")
    s = lax.axis_index("s")
    pltpu.sync_copy(zeros128.at[pl.ds(s * CPT, CPT)],
                    pacc.at[pl.ds(s * CPT, CPT)])
    plsc.subcore_barrier()

    @pl.when(c == 0)
    def _():
        pltpu.sync_copy(batch3.at[s], batchidx)

        def step(j, carry):
            pltpu.sync_copy(f_hbm.at[pl.ds(s * RPT + j * SB, SB)], rb)
            pltpu.sync_copy(rb, pacc.at[batchidx.at[j]], add=True)
            return carry
        lax.fori_loop(0, BPT, step, 0)

    plsc.subcore_barrier()

    @pl.when(c == 0)
    def _():
        pltpu.sync_copy(pacc.at[pl.ds(s * CPT, CPT)],
                        out.at[pl.ds(s * CPT, CPT)])


def _tc_first_body(dp_ref, x_ref, w_ref, g_ref, dis_ref):
    deg = dp_ref[0, :, 0:1] + dp_ref[1, :, 0:1] + 1.0
    dis = lax.rsqrt(deg)
    disb = jnp.broadcast_to(dis, (RB, D))
    g_ref[...] = jnp.dot(x_ref[...], w_ref[...],
                         preferred_element_type=jnp.float32) * disb
    dis_ref[...] = disb


_tc_first = pl.pallas_call(
    _tc_first_body,
    grid=(GRID,),
    in_specs=[
        pl.BlockSpec((NC, RB, D), lambda i: (0, i, 0)),
        pl.BlockSpec((RB, D), lambda i: (i, 0)),
        pl.BlockSpec((D, D), lambda i: (0, 0)),
    ],
    out_specs=[
        pl.BlockSpec((RB, D), lambda i: (i, 0)),
        pl.BlockSpec((RB, D), lambda i: (i, 0)),
    ],
    out_shape=[
        jax.ShapeDtypeStruct((NP, D), jnp.float32),
        jax.ShapeDtypeStruct((NP, D), jnp.float32),
    ],
)


def _tc_layer_body(p_ref, g_ref, dis_ref, b_ref, w_ref, o_ref):
    sagg = p_ref[0] + p_ref[1] + g_ref[...]
    f = jnp.maximum(dis_ref[...] * sagg + b_ref[...], 0.0)
    o_ref[...] = jnp.dot(f, w_ref[...],
                         preferred_element_type=jnp.float32) * dis_ref[...]


_tc_layer = pl.pallas_call(
    _tc_layer_body,
    grid=(GRID,),
    in_specs=[
        pl.BlockSpec((NC, RB, D), lambda i: (0, i, 0)),
        pl.BlockSpec((RB, D), lambda i: (i, 0)),
        pl.BlockSpec((RB, D), lambda i: (i, 0)),
        pl.BlockSpec((1, D), lambda i: (0, 0)),
        pl.BlockSpec((D, D), lambda i: (0, 0)),
    ],
    out_specs=pl.BlockSpec((RB, D), lambda i: (i, 0)),
    out_shape=jax.ShapeDtypeStruct((NP, D), jnp.float32),
)


def _tc_last_body(p_ref, g_ref, dis_ref, b_ref, o_ref):
    sagg = p_ref[0] + p_ref[1] + g_ref[...]
    o_ref[...] = jnp.maximum(dis_ref[...] * sagg + b_ref[...], 0.0)


_tc_last = pl.pallas_call(
    _tc_last_body,
    grid=(GRID,),
    in_specs=[
        pl.BlockSpec((NC, RB, D), lambda i: (0, i, 0)),
        pl.BlockSpec((RB, D), lambda i: (i, 0)),
        pl.BlockSpec((RB, D), lambda i: (i, 0)),
        pl.BlockSpec((1, D), lambda i: (0, 0)),
    ],
    out_specs=pl.BlockSpec((RB, D), lambda i: (i, 0)),
    out_shape=jax.ShapeDtypeStruct((NP, D), jnp.float32),
)


def _tc_final_body(pool_ref, cnt_ref, o_ref):
    cnt = cnt_ref[0:G, 0:1]
    o_ref[...] = pool_ref[0:G, :] / jnp.maximum(cnt, 1.0)


_tc_final = pl.pallas_call(
    _tc_final_body,
    in_specs=[
        pl.BlockSpec((CROWS, D), lambda: (0, 0)),
        pl.BlockSpec((CROWS, D), lambda: (0, 0)),
    ],
    out_specs=pl.BlockSpec((G, D), lambda: (0, 0)),
    out_shape=jax.ShapeDtypeStruct((G, D), jnp.float32),
)


def kernel(x, edge_index, batch, W0, b0, W1, b1, W2, b2, W3, b3, W4, b4):
    row = edge_index[0].astype(jnp.int32)
    col = edge_index[1].astype(jnp.int32)

    pad_e = EPAD - E
    pad_ar = jnp.arange(pad_e, dtype=jnp.int32)
    row_pad = jnp.concatenate([row, pad_ar % N])
    col_pad = jnp.concatenate([col, N + pad_ar % (NP - N)])
    row3 = row_pad.reshape(NW, TPT, SB)
    col3 = col_pad.reshape(NW, TPT, SB)

    pad_b = jnp.arange(NP - N, dtype=jnp.int32)
    batch_pad = jnp.concatenate([batch.astype(jnp.int32), G + pad_b % (CROWS - G)])
    batch3 = batch_pad.reshape(NS, BPT, SB)

    x_pad = jnp.concatenate([x, jnp.zeros((NP - N, D), jnp.float32)])
    zeros128 = jnp.zeros((NP, D), jnp.float32)
    ones128 = jnp.ones((SB, D), jnp.float32)

    degp, cnts = _sc_degrees(col3, batch3, ones128, zeros128)
    g, dis2d = _tc_first(degp, x_pad, W0)

    ws = [W1, W2, W3, W4]
    bs = [b0, b1, b2, b3]
    for t in range(4):
        p = _sc_aggregate(g, row3, col3, zeros128)
        g = _tc_layer(p, g, dis2d, bs[t].reshape(1, D), ws[t])
    p = _sc_aggregate(g, row3, col3, zeros128)
    f5 = _tc_last(p, g, dis2d, b4.reshape(1, D))

    pool = _sc_pool(f5, batch3, zeros128)
    return _tc_final(pool, cnts)


# speedup vs baseline: 1.1872x; 1.1872x over previous
import functools

import jax
import jax.numpy as jnp
from jax import lax
from jax.experimental import pallas as pl
from jax.experimental.pallas import tpu as pltpu
from jax.experimental.pallas import tpu_sc as plsc

N = 10000
E = 320000
D = 128
G = 128

NP = 10240
SB = 64
NC, NS = 2, 16
NW = NC * NS
TPT = 160
EPAD = NW * TPT * SB
RPT = NP // NS
CROWS = 256
CPT = CROWS // NS
BPT = NP // NS // SB

RB = 1024
GRID = NP // RB

_mesh = plsc.VectorSubcoreMesh(core_axis_name="c", subcore_axis_name="s")


@functools.partial(
    pl.kernel,
    out_type=[
        jax.ShapeDtypeStruct((NC, NP, D), jnp.float32),
        jax.ShapeDtypeStruct((CROWS, D), jnp.float32),
    ],
    mesh=_mesh,
    scratch_types=[
        pltpu.VMEM_SHARED((NP, D), jnp.float32),
        pltpu.VMEM_SHARED((CROWS, D), jnp.float32),
        pltpu.VMEM((TPT, SB), jnp.int32),
        pltpu.VMEM((BPT, SB), jnp.int32),
        pltpu.VMEM((SB, D), jnp.float32),
        pltpu.SemaphoreType.DMA,
        pltpu.SemaphoreType.DMA,
    ],
)
def _sc_degrees(col3, batch3, ones128, zeros128, deg_out, cnt_out,
                dacc, cacc, colidx, batchidx, ones_v, ssem0, ssem1):
    c = lax.axis_index("c")
    s = lax.axis_index("s")
    wid = c * NS + s
    pltpu.sync_copy(zeros128.at[pl.ds(s * RPT, RPT)], dacc.at[pl.ds(s * RPT, RPT)])
    pltpu.sync_copy(zeros128.at[pl.ds(s * CPT, CPT)], cacc.at[pl.ds(s * CPT, CPT)])
    pltpu.sync_copy(ones128, ones_v)
    pltpu.sync_copy(col3.at[wid], colidx)
    plsc.subcore_barrier()

    def deg_pair(k2, carry):
        k = 2 * k2
        pltpu.async_copy(ones_v, dacc.at[colidx.at[k]], ssem0, add=True)

        @pl.when(k2 > 0)
        def _():
            pltpu.make_async_copy(ones_v, dacc.at[colidx.at[k - 1]],
                                  ssem1).wait()
        pltpu.async_copy(ones_v, dacc.at[colidx.at[k + 1]], ssem1, add=True)
        pltpu.make_async_copy(ones_v, dacc.at[colidx.at[k]], ssem0).wait()
        return carry
    lax.fori_loop(0, TPT // 2, deg_pair, 0)
    pltpu.make_async_copy(ones_v, dacc.at[colidx.at[TPT - 1]], ssem1).wait()

    @pl.when(c == 0)
    def _():
        pltpu.sync_copy(batch3.at[s], batchidx)

        def cnt_step(j, carry):
            pltpu.sync_copy(ones_v, cacc.at[batchidx.at[j]], add=True)
            return carry
        lax.fori_loop(0, BPT, cnt_step, 0)

    plsc.subcore_barrier()
    pltpu.sync_copy(dacc.at[pl.ds(s * RPT, RPT)],
                    deg_out.at[c, pl.ds(s * RPT, RPT)])

    @pl.when(c == 0)
    def _():
        pltpu.sync_copy(cacc.at[pl.ds(s * CPT, CPT)],
                        cnt_out.at[pl.ds(s * CPT, CPT)])


@functools.partial(
    pl.kernel,
    out_type=jax.ShapeDtypeStruct((NC, NP, D), jnp.float32),
    mesh=_mesh,
    scratch_types=[
        pltpu.VMEM_SHARED((NP, D), jnp.float32),
        pltpu.VMEM((TPT // 4, SB), jnp.int32),
        pltpu.VMEM((TPT // 4, SB), jnp.int32),
        pltpu.VMEM((SB, D), jnp.float32),
        pltpu.VMEM((SB, D), jnp.float32),
        pltpu.VMEM((SB, D), jnp.float32),
        pltpu.VMEM((SB, D), jnp.float32),
        pltpu.SemaphoreType.DMA,
        pltpu.SemaphoreType.DMA,
        pltpu.SemaphoreType.DMA,
        pltpu.SemaphoreType.DMA,
    ],
)
def _sc_aggregate(g_hbm, row3, col3, zeros128, out,
                  accum, rowidx, colidx, gb0, gb1, gb2, gb3,
                  gsem0, gsem1, gsem2, gsem3):
    c = lax.axis_index("c")
    s = lax.axis_index("s")
    wid = c * NS + s
    QTR = TPT // 4
    gbs = (gb0, gb1, gb2, gb3)
    gsems = (gsem0, gsem1, gsem2, gsem3)
    pltpu.sync_copy(zeros128.at[pl.ds(s * RPT, RPT)],
                    accum.at[pl.ds(s * RPT, RPT)])
    plsc.subcore_barrier()

    for h in range(4):
        pltpu.sync_copy(row3.at[wid, pl.ds(h * QTR, QTR)], rowidx)
        pltpu.sync_copy(col3.at[wid, pl.ds(h * QTR, QTR)], colidx)
        for j in range(4):
            pltpu.async_copy(g_hbm.at[rowidx.at[j]], gbs[j], gsems[j])

        def quad(k4, carry):
            k = 4 * k4
            for j in range(4):
                pltpu.make_async_copy(g_hbm.at[rowidx.at[k + j]],
                                      gbs[j], gsems[j]).wait()
                pltpu.sync_copy(gbs[j], accum.at[colidx.at[k + j]], add=True)

                @pl.when(k + j + 4 < QTR)
                def _():
                    pltpu.async_copy(g_hbm.at[rowidx.at[k + j + 4]],
                                     gbs[j], gsems[j])
            return carry
        lax.fori_loop(0, QTR // 4, quad, 0)

    plsc.subcore_barrier()
    pltpu.sync_copy(accum.at[pl.ds(s * RPT, RPT)],
                    out.at[c, pl.ds(s * RPT, RPT)])


@functools.partial(
    pl.kernel,
    out_type=jax.ShapeDtypeStruct((CROWS, D), jnp.float32),
    mesh=_mesh,
    scratch_types=[
        pltpu.VMEM_SHARED((CROWS, D), jnp.float32),
        pltpu.VMEM((BPT, SB), jnp.int32),
        pltpu.VMEM((SB, D), jnp.float32),
    ],
)
def _sc_pool(f_hbm, batch3, zeros128, out, pacc, batchidx, rb):
    c = lax.axis_index("c")
    s = lax.axis_index("s")
    pltpu.sync_copy(zeros128.at[pl.ds(s * CPT, CPT)],
                    pacc.at[pl.ds(s * CPT, CPT)])
    plsc.subcore_barrier()

    @pl.when(c == 0)
    def _():
        pltpu.sync_copy(batch3.at[s], batchidx)

        def step(j, carry):
            pltpu.sync_copy(f_hbm.at[pl.ds(s * RPT + j * SB, SB)], rb)
            pltpu.sync_copy(rb, pacc.at[batchidx.at[j]], add=True)
            return carry
        lax.fori_loop(0, BPT, step, 0)

    plsc.subcore_barrier()

    @pl.when(c == 0)
    def _():
        pltpu.sync_copy(pacc.at[pl.ds(s * CPT, CPT)],
                        out.at[pl.ds(s * CPT, CPT)])


def _tc_first_body(dp_ref, x_ref, w_ref, g_ref, dis_ref):
    deg = dp_ref[0, :, 0:1] + dp_ref[1, :, 0:1] + 1.0
    dis = lax.rsqrt(deg)
    disb = jnp.broadcast_to(dis, (RB, D))
    g_ref[...] = jnp.dot(x_ref[...], w_ref[...],
                         preferred_element_type=jnp.float32) * disb
    dis_ref[...] = disb


_tc_first = pl.pallas_call(
    _tc_first_body,
    grid=(GRID,),
    in_specs=[
        pl.BlockSpec((NC, RB, D), lambda i: (0, i, 0)),
        pl.BlockSpec((RB, D), lambda i: (i, 0)),
        pl.BlockSpec((D, D), lambda i: (0, 0)),
    ],
    out_specs=[
        pl.BlockSpec((RB, D), lambda i: (i, 0)),
        pl.BlockSpec((RB, D), lambda i: (i, 0)),
    ],
    out_shape=[
        jax.ShapeDtypeStruct((NP, D), jnp.float32),
        jax.ShapeDtypeStruct((NP, D), jnp.float32),
    ],
)


def _tc_layer_body(p_ref, g_ref, dis_ref, b_ref, w_ref, o_ref):
    sagg = p_ref[0] + p_ref[1] + g_ref[...]
    f = jnp.maximum(dis_ref[...] * sagg + b_ref[...], 0.0)
    o_ref[...] = jnp.dot(f, w_ref[...],
                         preferred_element_type=jnp.float32) * dis_ref[...]


_tc_layer = pl.pallas_call(
    _tc_layer_body,
    grid=(GRID,),
    in_specs=[
        pl.BlockSpec((NC, RB, D), lambda i: (0, i, 0)),
        pl.BlockSpec((RB, D), lambda i: (i, 0)),
        pl.BlockSpec((RB, D), lambda i: (i, 0)),
        pl.BlockSpec((1, D), lambda i: (0, 0)),
        pl.BlockSpec((D, D), lambda i: (0, 0)),
    ],
    out_specs=pl.BlockSpec((RB, D), lambda i: (i, 0)),
    out_shape=jax.ShapeDtypeStruct((NP, D), jnp.float32),
)


def _tc_last_body(p_ref, g_ref, dis_ref, b_ref, o_ref):
    sagg = p_ref[0] + p_ref[1] + g_ref[...]
    o_ref[...] = jnp.maximum(dis_ref[...] * sagg + b_ref[...], 0.0)


_tc_last = pl.pallas_call(
    _tc_last_body,
    grid=(GRID,),
    in_specs=[
        pl.BlockSpec((NC, RB, D), lambda i: (0, i, 0)),
        pl.BlockSpec((RB, D), lambda i: (i, 0)),
        pl.BlockSpec((RB, D), lambda i: (i, 0)),
        pl.BlockSpec((1, D), lambda i: (0, 0)),
    ],
    out_specs=pl.BlockSpec((RB, D), lambda i: (i, 0)),
    out_shape=jax.ShapeDtypeStruct((NP, D), jnp.float32),
)


def _tc_final_body(pool_ref, cnt_ref, o_ref):
    cnt = cnt_ref[0:G, 0:1]
    o_ref[...] = pool_ref[0:G, :] / jnp.maximum(cnt, 1.0)


_tc_final = pl.pallas_call(
    _tc_final_body,
    in_specs=[
        pl.BlockSpec((CROWS, D), lambda: (0, 0)),
        pl.BlockSpec((CROWS, D), lambda: (0, 0)),
    ],
    out_specs=pl.BlockSpec((G, D), lambda: (0, 0)),
    out_shape=jax.ShapeDtypeStruct((G, D), jnp.float32),
)


def kernel(x, edge_index, batch, W0, b0, W1, b1, W2, b2, W3, b3, W4, b4):
    row = edge_index[0].astype(jnp.int32)
    col = edge_index[1].astype(jnp.int32)

    pad_e = EPAD - E
    pad_ar = jnp.arange(pad_e, dtype=jnp.int32)
    row_pad = jnp.concatenate([row, pad_ar % N])
    col_pad = jnp.concatenate([col, N + pad_ar % (NP - N)])
    row3 = row_pad.reshape(NW, TPT, SB)
    col3 = col_pad.reshape(NW, TPT, SB)

    pad_b = jnp.arange(NP - N, dtype=jnp.int32)
    batch_pad = jnp.concatenate([batch.astype(jnp.int32), G + pad_b % (CROWS - G)])
    batch3 = batch_pad.reshape(NS, BPT, SB)

    x_pad = jnp.concatenate([x, jnp.zeros((NP - N, D), jnp.float32)])
    zeros128 = jnp.zeros((NP, D), jnp.float32)
    ones128 = jnp.ones((SB, D), jnp.float32)

    degp, cnts = _sc_degrees(col3, batch3, ones128, zeros128)
    g, dis2d = _tc_first(degp, x_pad, W0)

    ws = [W1, W2, W3, W4]
    bs = [b0, b1, b2, b3]
    for t in range(4):
        p = _sc_aggregate(g, row3, col3, zeros128)
        g = _tc_layer(p, g, dis2d, bs[t].reshape(1, D), ws[t])
    p = _sc_aggregate(g, row3, col3, zeros128)
    f5 = _tc_last(p, g, dis2d, b4.reshape(1, D))

    pool = _sc_pool(f5, batch3, zeros128)
    return _tc_final(pool, cnts)


# speedup vs baseline: 1.1970x; 1.0083x over previous
import functools

import jax
import jax.numpy as jnp
from jax import lax
from jax.experimental import pallas as pl
from jax.experimental.pallas import tpu as pltpu
from jax.experimental.pallas import tpu_sc as plsc

N = 10000
E = 320000
D = 128
G = 128

NP = 10240
SB = 64
NC, NS = 2, 16
NW = NC * NS
TPT = 160
EPAD = NW * TPT * SB
RPT = NP // NS
CROWS = 256
CPT = CROWS // NS
BPT = NP // NW // SB

RB = 1024
GRID = NP // RB

_mesh = plsc.VectorSubcoreMesh(core_axis_name="c", subcore_axis_name="s")


@functools.partial(
    pl.kernel,
    out_type=[
        jax.ShapeDtypeStruct((NC, NP, D), jnp.float32),
        jax.ShapeDtypeStruct((NC, CROWS, D), jnp.float32),
    ],
    mesh=_mesh,
    scratch_types=[
        pltpu.VMEM_SHARED((NP, D), jnp.float32),
        pltpu.VMEM_SHARED((CROWS, D), jnp.float32),
        pltpu.VMEM((TPT, SB), jnp.int32),
        pltpu.VMEM((BPT, SB), jnp.int32),
        pltpu.VMEM((SB, D), jnp.float32),
        pltpu.SemaphoreType.DMA,
        pltpu.SemaphoreType.DMA,
    ],
)
def _sc_degrees(col3, batch3, ones128, zeros128, deg_out, cnt_out,
                dacc, cacc, colidx, batchidx, ones_v, ssem0, ssem1):
    c = lax.axis_index("c")
    s = lax.axis_index("s")
    wid = c * NS + s
    pltpu.sync_copy(zeros128.at[pl.ds(s * RPT, RPT)], dacc.at[pl.ds(s * RPT, RPT)])
    pltpu.sync_copy(zeros128.at[pl.ds(s * CPT, CPT)], cacc.at[pl.ds(s * CPT, CPT)])
    pltpu.sync_copy(ones128, ones_v)
    pltpu.sync_copy(col3.at[wid], colidx)
    plsc.subcore_barrier()

    def deg_pair(k2, carry):
        k = 2 * k2
        pltpu.async_copy(ones_v, dacc.at[colidx.at[k]], ssem0, add=True)

        @pl.when(k2 > 0)
        def _():
            pltpu.make_async_copy(ones_v, dacc.at[colidx.at[k - 1]],
                                  ssem1).wait()
        pltpu.async_copy(ones_v, dacc.at[colidx.at[k + 1]], ssem1, add=True)
        pltpu.make_async_copy(ones_v, dacc.at[colidx.at[k]], ssem0).wait()
        return carry
    lax.fori_loop(0, TPT // 2, deg_pair, 0)
    pltpu.make_async_copy(ones_v, dacc.at[colidx.at[TPT - 1]], ssem1).wait()

    pltpu.sync_copy(batch3.at[wid], batchidx)

    def cnt_step(j, carry):
        pltpu.sync_copy(ones_v, cacc.at[batchidx.at[j]], add=True)
        return carry
    lax.fori_loop(0, BPT, cnt_step, 0)

    plsc.subcore_barrier()
    pltpu.sync_copy(dacc.at[pl.ds(s * RPT, RPT)],
                    deg_out.at[c, pl.ds(s * RPT, RPT)])
    pltpu.sync_copy(cacc.at[pl.ds(s * CPT, CPT)],
                    cnt_out.at[c, pl.ds(s * CPT, CPT)])


@functools.partial(
    pl.kernel,
    out_type=jax.ShapeDtypeStruct((NC, NP, D), jnp.float32),
    mesh=_mesh,
    scratch_types=[
        pltpu.VMEM_SHARED((NP, D), jnp.float32),
        pltpu.VMEM((TPT // 4, SB), jnp.int32),
        pltpu.VMEM((TPT // 4, SB), jnp.int32),
        pltpu.VMEM((SB, D), jnp.float32),
        pltpu.VMEM((SB, D), jnp.float32),
        pltpu.VMEM((SB, D), jnp.float32),
        pltpu.VMEM((SB, D), jnp.float32),
        pltpu.SemaphoreType.DMA,
        pltpu.SemaphoreType.DMA,
        pltpu.SemaphoreType.DMA,
        pltpu.SemaphoreType.DMA,
    ],
)
def _sc_aggregate(g_hbm, row3, col3, zeros128, out,
                  accum, rowidx, colidx, gb0, gb1, gb2, gb3,
                  gsem0, gsem1, gsem2, gsem3):
    c = lax.axis_index("c")
    s = lax.axis_index("s")
    wid = c * NS + s
    QTR = TPT // 4
    gbs = (gb0, gb1, gb2, gb3)
    gsems = (gsem0, gsem1, gsem2, gsem3)
    pltpu.sync_copy(zeros128.at[pl.ds(s * RPT, RPT)],
                    accum.at[pl.ds(s * RPT, RPT)])
    plsc.subcore_barrier()

    for h in range(4):
        pltpu.sync_copy(row3.at[wid, pl.ds(h * QTR, QTR)], rowidx)
        pltpu.sync_copy(col3.at[wid, pl.ds(h * QTR, QTR)], colidx)
        for j in range(4):
            pltpu.async_copy(g_hbm.at[rowidx.at[j]], gbs[j], gsems[j])

        def quad(k4, carry):
            k = 4 * k4
            for j in range(4):
                pltpu.make_async_copy(g_hbm.at[rowidx.at[k + j]],
                                      gbs[j], gsems[j]).wait()
                pltpu.sync_copy(gbs[j], accum.at[colidx.at[k + j]], add=True)

                @pl.when(k + j + 4 < QTR)
                def _():
                    pltpu.async_copy(g_hbm.at[rowidx.at[k + j + 4]],
                                     gbs[j], gsems[j])
            return carry
        lax.fori_loop(0, QTR // 4, quad, 0)

    plsc.subcore_barrier()
    pltpu.sync_copy(accum.at[pl.ds(s * RPT, RPT)],
                    out.at[c, pl.ds(s * RPT, RPT)])


@functools.partial(
    pl.kernel,
    out_type=jax.ShapeDtypeStruct((NC, CROWS, D), jnp.float32),
    mesh=_mesh,
    scratch_types=[
        pltpu.VMEM_SHARED((CROWS, D), jnp.float32),
        pltpu.VMEM((BPT, SB), jnp.int32),
        pltpu.VMEM((SB, D), jnp.float32),
    ],
)
def _sc_pool(f_hbm, batch3, zeros128, out, pacc, batchidx, rb):
    c = lax.axis_index("c")
    s = lax.axis_index("s")
    wid = c * NS + s
    pltpu.sync_copy(zeros128.at[pl.ds(s * CPT, CPT)],
                    pacc.at[pl.ds(s * CPT, CPT)])
    plsc.subcore_barrier()

    pltpu.sync_copy(batch3.at[wid], batchidx)

    def step(j, carry):
        pltpu.sync_copy(f_hbm.at[pl.ds(wid * (BPT * SB) + j * SB, SB)], rb)
        pltpu.sync_copy(rb, pacc.at[batchidx.at[j]], add=True)
        return carry
    lax.fori_loop(0, BPT, step, 0)

    plsc.subcore_barrier()
    pltpu.sync_copy(pacc.at[pl.ds(s * CPT, CPT)],
                    out.at[c, pl.ds(s * CPT, CPT)])


def _tc_mm0_body(x_ref, w_ref, u_ref):
    u_ref[...] = jnp.dot(x_ref[...], w_ref[...],
                         preferred_element_type=jnp.float32)


_tc_mm0 = pl.pallas_call(
    _tc_mm0_body,
    grid=(GRID,),
    in_specs=[
        pl.BlockSpec((RB, D), lambda i: (i, 0)),
        pl.BlockSpec((D, D), lambda i: (0, 0)),
    ],
    out_specs=pl.BlockSpec((RB, D), lambda i: (i, 0)),
    out_shape=jax.ShapeDtypeStruct((NP, D), jnp.float32),
)


def _tc_g0_body(dp_ref, u_ref, g_ref, dis_ref):
    deg = dp_ref[0, :, 0:1] + dp_ref[1, :, 0:1] + 1.0
    dis = lax.rsqrt(deg)
    disb = jnp.broadcast_to(dis, (RB, D))
    g_ref[...] = u_ref[...] * disb
    dis_ref[...] = disb


_tc_g0 = pl.pallas_call(
    _tc_g0_body,
    grid=(GRID,),
    in_specs=[
        pl.BlockSpec((NC, RB, D), lambda i: (0, i, 0)),
        pl.BlockSpec((RB, D), lambda i: (i, 0)),
    ],
    out_specs=[
        pl.BlockSpec((RB, D), lambda i: (i, 0)),
        pl.BlockSpec((RB, D), lambda i: (i, 0)),
    ],
    out_shape=[
        jax.ShapeDtypeStruct((NP, D), jnp.float32),
        jax.ShapeDtypeStruct((NP, D), jnp.float32),
    ],
)


def _tc_layer_body(p_ref, g_ref, dis_ref, b_ref, w_ref, o_ref):
    sagg = p_ref[0] + p_ref[1] + g_ref[...]
    f = jnp.maximum(dis_ref[...] * sagg + b_ref[...], 0.0)
    o_ref[...] = jnp.dot(f, w_ref[...],
                         preferred_element_type=jnp.float32) * dis_ref[...]


_tc_layer = pl.pallas_call(
    _tc_layer_body,
    grid=(GRID,),
    in_specs=[
        pl.BlockSpec((NC, RB, D), lambda i: (0, i, 0)),
        pl.BlockSpec((RB, D), lambda i: (i, 0)),
        pl.BlockSpec((RB, D), lambda i: (i, 0)),
        pl.BlockSpec((1, D), lambda i: (0, 0)),
        pl.BlockSpec((D, D), lambda i: (0, 0)),
    ],
    out_specs=pl.BlockSpec((RB, D), lambda i: (i, 0)),
    out_shape=jax.ShapeDtypeStruct((NP, D), jnp.float32),
)


def _tc_last_body(p_ref, g_ref, dis_ref, b_ref, o_ref):
    sagg = p_ref[0] + p_ref[1] + g_ref[...]
    o_ref[...] = jnp.maximum(dis_ref[...] * sagg + b_ref[...], 0.0)


_tc_last = pl.pallas_call(
    _tc_last_body,
    grid=(GRID,),
    in_specs=[
        pl.BlockSpec((NC, RB, D), lambda i: (0, i, 0)),
        pl.BlockSpec((RB, D), lambda i: (i, 0)),
        pl.BlockSpec((RB, D), lambda i: (i, 0)),
        pl.BlockSpec((1, D), lambda i: (0, 0)),
    ],
    out_specs=pl.BlockSpec((RB, D), lambda i: (i, 0)),
    out_shape=jax.ShapeDtypeStruct((NP, D), jnp.float32),
)


def _tc_final_body(pool_ref, cnt_ref, o_ref):
    cnt = cnt_ref[0, 0:G, 0:1] + cnt_ref[1, 0:G, 0:1]
    pool = pool_ref[0, 0:G, :] + pool_ref[1, 0:G, :]
    o_ref[...] = pool / jnp.maximum(cnt, 1.0)


_tc_final = pl.pallas_call(
    _tc_final_body,
    in_specs=[
        pl.BlockSpec((NC, CROWS, D), lambda: (0, 0, 0)),
        pl.BlockSpec((NC, CROWS, D), lambda: (0, 0, 0)),
    ],
    out_specs=pl.BlockSpec((G, D), lambda: (0, 0)),
    out_shape=jax.ShapeDtypeStruct((G, D), jnp.float32),
)


def kernel(x, edge_index, batch, W0, b0, W1, b1, W2, b2, W3, b3, W4, b4):
    row = edge_index[0].astype(jnp.int32)
    col = edge_index[1].astype(jnp.int32)

    pad_e = EPAD - E
    pad_ar = jnp.arange(pad_e, dtype=jnp.int32)
    row_pad = jnp.concatenate([row, pad_ar % N])
    col_pad = jnp.concatenate([col, N + pad_ar % (NP - N)])
    row3 = row_pad.reshape(NW, TPT, SB)
    col3 = col_pad.reshape(NW, TPT, SB)

    pad_b = jnp.arange(NP - N, dtype=jnp.int32)
    batch_pad = jnp.concatenate([batch.astype(jnp.int32), G + pad_b % (CROWS - G)])
    batch3 = batch_pad.reshape(NW, BPT, SB)

    x_pad = jnp.concatenate([x, jnp.zeros((NP - N, D), jnp.float32)])
    zeros128 = jnp.zeros((NP, D), jnp.float32)
    ones128 = jnp.ones((SB, D), jnp.float32)

    u0 = _tc_mm0(x_pad, W0)
    degp, cnts = _sc_degrees(col3, batch3, ones128, zeros128)
    g, dis2d = _tc_g0(degp, u0)

    ws = [W1, W2, W3, W4]
    bs = [b0, b1, b2, b3]
    for t in range(4):
        p = _sc_aggregate(g, row3, col3, zeros128)
        g = _tc_layer(p, g, dis2d, bs[t].reshape(1, D), ws[t])
    p = _sc_aggregate(g, row3, col3, zeros128)
    f5 = _tc_last(p, g, dis2d, b4.reshape(1, D))

    pool = _sc_pool(f5, batch3, zeros128)
    return _tc_final(pool, cnts)
